# baseline (speedup 1.0000x reference)
import jax
import jax.numpy as jnp
from jax import lax
from jax.experimental import pallas as pl
from jax.experimental.pallas import tpu as pltpu
from jax.experimental.pallas import tpu_sc as plsc

B, S, D = 16, 2048, 1024
NC, NS = 2, 16
NW = NC * NS
DW = D // NW
L = 16
CHUNK = 512


def _take_mean_body(x_hbm, len_hbm, ilen_hbm, out_hbm, len_v, ilen_v, buf, obuf):
    c = lax.axis_index("c")
    s = lax.axis_index("s")
    wid = s * NC + c
    d0 = wid * DW

    pltpu.sync_copy(len_hbm, len_v)
    pltpu.sync_copy(ilen_hbm, ilen_v)
    len_vec = len_v[...]
    ilen_vec = ilen_v[...]

    for b in range(B):
        n = len_vec[b]
        nchunks = (n + CHUNK - 1) // CHUNK

        def chunk_body(k, accs, b=b):
            a0, a1 = accs
            s0 = k * CHUNK
            pltpu.sync_copy(x_hbm.at[b, pl.ds(s0, CHUNK), pl.ds(d0, DW)], buf)
            m = jnp.minimum(CHUNK, n - s0)

            def row_body(r, accs2):
                b0, b1 = accs2
                return (b0 + buf[r, pl.ds(0, L)], b1 + buf[r, pl.ds(L, L)])

            return lax.fori_loop(0, m, row_body, (a0, a1))

        zero = jnp.zeros((L,), jnp.float32)
        a0, a1 = lax.fori_loop(0, nchunks, chunk_body, (zero, zero))
        inv = ilen_vec[b]
        obuf[b, pl.ds(0, L)] = a0 * inv
        obuf[b, pl.ds(L, L)] = a1 * inv

    pltpu.sync_copy(obuf, out_hbm.at[:, pl.ds(d0, DW)])


_mesh = plsc.VectorSubcoreMesh(
    core_axis_name="c", subcore_axis_name="s", num_cores=NC, num_subcores=NS
)

_take_mean_sc = pl.kernel(
    _take_mean_body,
    out_type=jax.ShapeDtypeStruct((B, D), jnp.float32),
    mesh=_mesh,
    scratch_types=[
        pltpu.VMEM((L,), jnp.int32),
        pltpu.VMEM((L,), jnp.float32),
        pltpu.VMEM((CHUNK, DW), jnp.float32),
        pltpu.VMEM((B, DW), jnp.float32),
    ],
    compiler_params=pltpu.CompilerParams(use_tc_tiling_on_sc=False),
)


def kernel(padded_input, lengths):
    lengths_i32 = lengths.astype(jnp.int32)
    lengths_f32 = lengths_i32.astype(jnp.float32)
    means = _take_mean_sc(padded_input, lengths_i32, 1.0 / lengths_f32)
    return jnp.concatenate([means, lengths_f32[:, None]], axis=-1)


# speedup vs baseline: 1.2954x; 1.2954x over previous
import jax
import jax.numpy as jnp
from jax import lax
from jax.experimental import pallas as pl
from jax.experimental.pallas import tpu as pltpu
from jax.experimental.pallas import tpu_sc as plsc

B, S, D = 16, 2048, 1024
NC, NS = 2, 16
NW = NC * NS
DW = D // NW
L = 16
CHUNK = 512
KMAX = S // CHUNK


def _take_mean_body(x_hbm, len_hbm, ilen_hbm, out_hbm,
                    len_v, ilen_v, buf0, buf1, obuf, sem0, sem1):
    c = lax.axis_index("c")
    s = lax.axis_index("s")
    wid = s * NC + c
    d0 = wid * DW

    pltpu.sync_copy(len_hbm, len_v)
    pltpu.sync_copy(ilen_hbm, ilen_v)
    len_vec = len_v[...]
    ilen_vec = ilen_v[...]

    bufs = (buf0, buf1)
    sems = (sem0, sem1)
    slots = [(b, k) for b in range(B) for k in range(KMAX)]

    def src(b, k):
        return x_hbm.at[b, pl.ds(k * CHUNK, CHUNK), pl.ds(d0, DW)]

    def guard(b, k):
        return len_vec[b] > (k * CHUNK)

    pltpu.async_copy(src(0, 0), bufs[0], sems[0])

    zero = jnp.zeros((L,), jnp.float32)
    accs = None
    for i, (b, k) in enumerate(slots):
        if k == 0:
            accs = [zero] * 8
        if i + 1 < len(slots):
            b2, k2 = slots[i + 1]
            j = (i + 1) % 2

            @pl.when(guard(b2, k2))
            def _issue(b2=b2, k2=k2, j=j):
                pltpu.async_copy(src(b2, k2), bufs[j], sems[j])

        j = i % 2
        if k > 0:

            @pl.when(guard(b, k))
            def _wait(b=b, k=k, j=j):
                pltpu.make_async_copy(src(b, k), bufs[j], sems[j]).wait()
        else:
            pltpu.make_async_copy(src(b, k), bufs[j], sems[j]).wait()

        n = len_vec[b]
        m = jnp.clip(n - k * CHUNK, 0, CHUNK)
        m8 = (m + 7) & ~7
        buf = bufs[j]

        def zero_body(r, carry, buf=buf):
            buf[r, pl.ds(0, L)] = zero
            buf[r, pl.ds(L, L)] = zero
            return carry

        lax.fori_loop(m, m8, zero_body, 0)

        def acc_body(t, a, buf=buf):
            a = list(a)
            r = t * 8
            for u in range(8):
                a[(2 * u) % 8] += buf[r + u, pl.ds(0, L)]
                a[(2 * u + 1) % 8] += buf[r + u, pl.ds(L, L)]
            return tuple(a)

        accs = list(lax.fori_loop(0, m8 // 8, acc_body, tuple(accs)))

        if k == KMAX - 1:
            inv = ilen_vec[b]
            lo = (accs[0] + accs[2]) + (accs[4] + accs[6])
            hi = (accs[1] + accs[3]) + (accs[5] + accs[7])
            obuf[b, pl.ds(0, L)] = lo * inv
            obuf[b, pl.ds(L, L)] = hi * inv

    pltpu.sync_copy(obuf, out_hbm.at[:, pl.ds(d0, DW)])


_mesh = plsc.VectorSubcoreMesh(
    core_axis_name="c", subcore_axis_name="s", num_cores=NC, num_subcores=NS
)

_take_mean_sc = pl.kernel(
    _take_mean_body,
    out_type=jax.ShapeDtypeStruct((B, D), jnp.float32),
    mesh=_mesh,
    scratch_types=[
        pltpu.VMEM((L,), jnp.int32),
        pltpu.VMEM((L,), jnp.float32),
        pltpu.VMEM((CHUNK, DW), jnp.float32),
        pltpu.VMEM((CHUNK, DW), jnp.float32),
        pltpu.VMEM((B, DW), jnp.float32),
        pltpu.SemaphoreType.DMA,
        pltpu.SemaphoreType.DMA,
    ],
    compiler_params=pltpu.CompilerParams(use_tc_tiling_on_sc=False),
)


def kernel(padded_input, lengths):
    lengths_i32 = lengths.astype(jnp.int32)
    lengths_f32 = lengths_i32.astype(jnp.float32)
    means = _take_mean_sc(padded_input, lengths_i32, 1.0 / lengths_f32)
    return jnp.concatenate([means, lengths_f32[:, None]], axis=-1)


# speedup vs baseline: 3.3351x; 2.5745x over previous
import jax
import jax.numpy as jnp
from jax import lax
from jax.experimental import pallas as pl
from jax.experimental.pallas import tpu as pltpu
from jax.experimental.pallas import tpu_sc as plsc

B, S, D = 16, 2048, 1024
NC, NS = 2, 16
NSTRIPE = 8
SW = D // NSTRIPE
NG = 4
GB = B // NG
L = 16
CHUNK = 256
KMAX = S // CHUNK


def _take_mean_body(x_hbm, len_hbm, ilen_hbm, out_hbm,
                    len_v, ilen_v, buf0, buf1, obuf, sem0, sem1):
    c = lax.axis_index("c")
    s = lax.axis_index("s")
    st = s % NSTRIPE
    g = 2 * (s // NSTRIPE) + c
    d0 = pl.multiple_of(st * SW, SW)

    pltpu.sync_copy(len_hbm, len_v)
    pltpu.sync_copy(ilen_hbm, ilen_v)
    len_vec = len_v[...]
    ilen_vec = ilen_v[...]

    def pick(vec, bb):
        r = vec[3 * GB + bb]
        r = jnp.where(g == 2, vec[2 * GB + bb], r)
        r = jnp.where(g == 1, vec[1 * GB + bb], r)
        return jnp.where(g == 0, vec[0 * GB + bb], r)

    ns = [pick(len_vec, bb) for bb in range(GB)]
    invs = [pick(ilen_vec, bb) for bb in range(GB)]

    bufs = (buf0, buf1)
    sems = (sem0, sem1)
    slots = [(bb, k) for bb in range(GB) for k in range(KMAX)]

    def src(bb, k):
        return x_hbm.at[g * GB + bb, pl.ds(k * CHUNK, CHUNK), pl.ds(d0, SW)]

    def guard(bb, k):
        return ns[bb] > (k * CHUNK)

    pltpu.async_copy(src(0, 0), bufs[0], sems[0])

    zero = jnp.zeros((L,), jnp.float32)
    accs = None
    for i, (bb, k) in enumerate(slots):
        if k == 0:
            accs = [zero] * 8
        if i + 1 < len(slots):
            bb2, k2 = slots[i + 1]
            j = (i + 1) % 2

            @pl.when(guard(bb2, k2))
            def _issue(bb2=bb2, k2=k2, j=j):
                pltpu.async_copy(src(bb2, k2), bufs[j], sems[j])

        j = i % 2
        if k > 0:

            @pl.when(guard(bb, k))
            def _wait(bb=bb, k=k, j=j):
                pltpu.make_async_copy(src(bb, k), bufs[j], sems[j]).wait()
        else:
            pltpu.make_async_copy(src(bb, k), bufs[j], sems[j]).wait()

        m = jnp.clip(ns[bb] - k * CHUNK, 0, CHUNK)
        m4 = (m + 3) & ~3
        buf = bufs[j]

        def zero_body(r, carry, buf=buf):
            for v in range(8):
                buf[r, pl.ds(v * L, L)] = zero
            return carry

        lax.fori_loop(m, m4, zero_body, 0)

        def acc_body(t, a, buf=buf):
            a = list(a)
            r = t * 4
            for rr in range(4):
                for v in range(8):
                    a[v] += buf[r + rr, pl.ds(v * L, L)]
            return tuple(a)

        accs = list(lax.fori_loop(0, m4 // 4, acc_body, tuple(accs)))

        if k == KMAX - 1:
            for v in range(8):
                obuf[bb, pl.ds(v * L, L)] = accs[v] * invs[bb]

    pltpu.sync_copy(obuf, out_hbm.at[g, :, pl.ds(d0, SW)])


_mesh = plsc.VectorSubcoreMesh(
    core_axis_name="c", subcore_axis_name="s", num_cores=NC, num_subcores=NS
)

_take_mean_sc = pl.kernel(
    _take_mean_body,
    out_type=jax.ShapeDtypeStruct((NG, GB, D), jnp.float32),
    mesh=_mesh,
    scratch_types=[
        pltpu.VMEM((L,), jnp.int32),
        pltpu.VMEM((L,), jnp.float32),
        pltpu.VMEM((CHUNK, SW), jnp.float32),
        pltpu.VMEM((CHUNK, SW), jnp.float32),
        pltpu.VMEM((GB, SW), jnp.float32),
        pltpu.SemaphoreType.DMA,
        pltpu.SemaphoreType.DMA,
    ],
)


def kernel(padded_input, lengths):
    lengths_i32 = lengths.astype(jnp.int32)
    lengths_f32 = lengths_i32.astype(jnp.float32)
    means = _take_mean_sc(padded_input, lengths_i32, 1.0 / lengths_f32)
    means = means.reshape(B, D)
    return jnp.concatenate([means, lengths_f32[:, None]], axis=-1)


# speedup vs baseline: 3.6348x; 1.0899x over previous
import jax
import jax.numpy as jnp
from jax import lax
from jax.experimental import pallas as pl
from jax.experimental.pallas import tpu as pltpu
from jax.experimental.pallas import tpu_sc as plsc

B, S, D = 16, 2048, 1024
NC, NS = 2, 16
NSTRIPE = 8
SW = D // NSTRIPE
NG = 4
GB = B // NG
L = 16
CHUNK = 128

SNAKE = [0, 7, 8, 15, 1, 6, 9, 14, 2, 5, 10, 13, 3, 4, 11, 12]


def _take_mean_body(x_hbm, bidx_hbm, len_hbm, ilen_hbm, out_hbm,
                    bidx_v, len_v, ilen_v, bufh0, bufh1, buf0, buf1, obuf,
                    semh0, semh1, sem0, sem1):
    c = lax.axis_index("c")
    s = lax.axis_index("s")
    st = s % NSTRIPE
    g = 2 * (s // NSTRIPE) + c
    d0 = pl.multiple_of(st * SW, SW)

    pltpu.sync_copy(bidx_hbm, bidx_v)
    pltpu.sync_copy(len_hbm, len_v)
    pltpu.sync_copy(ilen_hbm, ilen_v)
    bidx_vec = bidx_v[...]
    len_vec = len_v[...]
    ilen_vec = ilen_v[...]

    def pick(vec, bb):
        r = vec[3 * GB + bb]
        r = jnp.where(g == 2, vec[2 * GB + bb], r)
        r = jnp.where(g == 1, vec[1 * GB + bb], r)
        return jnp.where(g == 0, vec[0 * GB + bb], r)

    bs = [pick(bidx_vec, bb) for bb in range(GB)]
    ns = [pick(len_vec, bb) for bb in range(GB)]
    invs = [pick(ilen_vec, bb) for bb in range(GB)]
    nchs = [(n + CHUNK - 1) // CHUNK for n in ns]

    bufhs = (bufh0, bufh1)
    semhs = (semh0, semh1)

    def src(bb, k):
        return x_hbm.at[bs[bb], pl.ds(k * CHUNK, CHUNK), pl.ds(d0, SW)]

    zero = jnp.zeros((L,), jnp.float32)

    def make_acc(buf, bb, k):
        def run(accs):
            m = jnp.clip(ns[bb] - k * CHUNK, 0, CHUNK)
            m8 = (m + 7) & ~7

            def zero_body(r, carry):
                for v in range(8):
                    buf[r, pl.ds(v * L, L)] = zero
                return carry

            lax.fori_loop(m, m8, zero_body, 0)

            def acc_body(t, a):
                a = list(a)
                r = t * 8
                for rr in range(8):
                    for v in range(8):
                        a[v] += buf[r + rr, pl.ds(v * L, L)]
                return tuple(a)

            return list(lax.fori_loop(0, m8 // 8, acc_body, tuple(accs)))

        return run

    pltpu.async_copy(src(0, 0), bufhs[0], semhs[0])

    for bb in range(GB):
        hb, hs = bufhs[bb % 2], semhs[bb % 2]
        n, nch = ns[bb], nchs[bb]

        @pl.when(nch > 1)
        def _issue1(bb=bb):
            pltpu.async_copy(src(bb, 1), buf0, sem0)

        if bb + 1 < GB:
            pltpu.async_copy(src(bb + 1, 0), bufhs[(bb + 1) % 2],
                             semhs[(bb + 1) % 2])

        pltpu.make_async_copy(src(bb, 0), hb, hs).wait()
        accs = make_acc(hb, bb, 0)([zero] * 8)

        def pair_body(t, a, bb=bb, nch=nch):
            k0 = 1 + 2 * t
            k1 = 2 + 2 * t

            @pl.when(k0 + 1 < nch)
            def _issue_k1(bb=bb, k0=k0):
                pltpu.async_copy(src(bb, k0 + 1), buf1, sem1)

            pltpu.make_async_copy(src(bb, k0), buf0, sem0).wait()
            a = make_acc(buf0, bb, k0)(a)

            @pl.when(k1 + 1 < nch)
            def _issue_k2(bb=bb, k1=k1):
                pltpu.async_copy(src(bb, k1 + 1), buf0, sem0)

            @pl.when(k1 < nch)
            def _wait_k1(bb=bb, k1=k1):
                pltpu.make_async_copy(src(bb, k1), buf1, sem1).wait()

            a = make_acc(buf1, bb, k1)(a)
            return tuple(a)

        npairs = (nch - 1 + 1) // 2
        accs = list(lax.fori_loop(0, npairs, pair_body, tuple(accs)))

        for v in range(8):
            obuf[bb, pl.ds(v * L, L)] = accs[v] * invs[bb]

    pltpu.sync_copy(obuf, out_hbm.at[g, :, pl.ds(d0, SW)])


_mesh = plsc.VectorSubcoreMesh(
    core_axis_name="c", subcore_axis_name="s", num_cores=NC, num_subcores=NS
)

_take_mean_sc = pl.kernel(
    _take_mean_body,
    out_type=jax.ShapeDtypeStruct((NG, GB, D), jnp.float32),
    mesh=_mesh,
    scratch_types=[
        pltpu.VMEM((L,), jnp.int32),
        pltpu.VMEM((L,), jnp.int32),
        pltpu.VMEM((L,), jnp.float32),
        pltpu.VMEM((CHUNK, SW), jnp.float32),
        pltpu.VMEM((CHUNK, SW), jnp.float32),
        pltpu.VMEM((CHUNK, SW), jnp.float32),
        pltpu.VMEM((CHUNK, SW), jnp.float32),
        pltpu.VMEM((GB, SW), jnp.float32),
        pltpu.SemaphoreType.DMA,
        pltpu.SemaphoreType.DMA,
        pltpu.SemaphoreType.DMA,
        pltpu.SemaphoreType.DMA,
    ],
)


def kernel(padded_input, lengths):
    lengths_i32 = lengths.astype(jnp.int32)
    lengths_f32 = lengths_i32.astype(jnp.float32)
    order = jnp.argsort(-lengths_i32)
    bidx = order[jnp.array(SNAKE, dtype=jnp.int32)]
    larr = lengths_i32[bidx]
    ilarr = 1.0 / lengths_f32[bidx]
    means_p = _take_mean_sc(padded_input, bidx, larr, ilarr)
    means = means_p.reshape(B, D)[jnp.argsort(bidx)]
    return jnp.concatenate([means, lengths_f32[:, None]], axis=-1)
